# baseline (speedup 1.0000x reference)
import functools

import jax
import jax.numpy as jnp
from jax import lax
from jax.experimental import pallas as pl
from jax.experimental.pallas import tpu as pltpu
from jax.experimental.pallas import tpu_sc as plsc

V_SIZE = 1_000_000
E = 32
L_SEQ = 200
B_BATCH = 4096
B_TOTAL = B_BATCH * L_SEQ
ROWS4 = V_SIZE // 4

NUM_CORES = 2
NUM_SUBCORES = 16
NW = NUM_CORES * NUM_SUBCORES

_mesh = plsc.VectorSubcoreMesh(core_axis_name="c", subcore_axis_name="s")


def _iota16():
    return jax.lax.broadcasted_iota(jnp.int32, (16,), 0)


def _transpose_block(in_v, out_v, in_row_base, out_col_base, n_tc):
    iota = _iota16()
    t4 = []
    tmod = []
    src_cols = []
    for tc in range(n_tc):
        t = iota + (16 * tc)
        t4.append(jax.lax.shift_right_logical(t, 2) + in_row_base)
        tmod.append(jax.lax.shift_left(jax.lax.bitwise_and(t, 3), 5))
        src_cols.append(t + out_col_base)

    def body(e, carry):
        e_splat = jnp.full((16,), e, jnp.int32)
        for tc in range(n_tc):
            vals = plsc.load_gather(in_v, [e_splat, src_cols[tc]])
            plsc.store_scatter(out_v, [t4[tc], tmod[tc] + e_splat], vals)
        return carry

    lax.fori_loop(0, E, body, 0)


def _untranspose_block(in_v, out_v, in_row_base, out_col_base, n_tc):
    iota = _iota16()
    t4 = []
    tmod = []
    dst_cols = []
    for tc in range(n_tc):
        t = iota + (16 * tc)
        t4.append(jax.lax.shift_right_logical(t, 2) + in_row_base)
        tmod.append(jax.lax.shift_left(jax.lax.bitwise_and(t, 3), 5))
        dst_cols.append(t + out_col_base)

    def body(e, carry):
        e_splat = jnp.full((16,), e, jnp.int32)
        for tc in range(n_tc):
            vals = plsc.load_gather(in_v, [t4[tc], tmod[tc] + e_splat])
            plsc.store_scatter(out_v, [e_splat, dst_cols[tc]], vals)
        return carry

    lax.fori_loop(0, E, body, 0)


A_BLOCKS = 1953
A_BPW = 62


@functools.partial(
    pl.kernel,
    mesh=_mesh,
    out_type=jax.ShapeDtypeStruct((ROWS4, 128), jnp.float32),
    scratch_types=[
        pltpu.VMEM((2, E, 512), jnp.float32),
        pltpu.VMEM((2, 128, 128), jnp.float32),
        pltpu.SemaphoreType.DMA,
        pltpu.SemaphoreType.DMA,
        pltpu.SemaphoreType.DMA,
        pltpu.SemaphoreType.DMA,
    ],
    compiler_params=pltpu.CompilerParams(needs_layout_passes=False),
)
def _reformat_table(vt_hbm, tail_hbm, out_hbm, in_v, out_v, gi0, gi1, wo0, wo1):
    gsem = (gi0, gi1)
    wsem = (wo0, wo1)
    wid = lax.axis_index("s") * NUM_CORES + lax.axis_index("c")

    def blk_of(k):
        return k * NW + wid

    def start_in(k, b):
        pltpu.async_copy(
            vt_hbm.at[0, :, pl.ds(blk_of(k) * 512, 512)], in_v.at[b], gsem[b])

    @pl.when(blk_of(0) < A_BLOCKS)
    def _():
        start_in(0, 0)

    @pl.loop(0, A_BPW // 2)
    def _pair(p):
        for b in range(2):
            k = p * 2 + b
            blk = blk_of(k)

            @pl.when(blk < A_BLOCKS)
            def _():
                pltpu.make_async_copy(
                    vt_hbm.at[0, :, pl.ds(0, 512)], in_v.at[b],
                    gsem[b]).wait()

                @pl.when(blk_of(k + 1) < A_BLOCKS)
                def _():
                    start_in(k + 1, 1 - b)

                @pl.when(k >= 2)
                def _():
                    pltpu.make_async_copy(
                        out_v.at[b], out_hbm.at[pl.ds(0, 128)],
                        wsem[b]).wait()

                @pl.loop(0, 4)
                def _col(c):
                    _transpose_block(in_v.at[b], out_v.at[b], c * 32,
                                     c * 128, 8)

                pltpu.async_copy(
                    out_v.at[b], out_hbm.at[pl.ds(blk * 128, 128)], wsem[b])

    for b in range(2):
        pltpu.make_async_copy(
            out_v.at[b], out_hbm.at[pl.ds(0, 128)], wsem[b]).wait()

    @pl.when(wid == NW - 1)
    def _():
        pltpu.sync_copy(tail_hbm, out_v.at[0, pl.ds(0, 16)])
        pltpu.sync_copy(out_v.at[0, pl.ds(0, 16)],
                        out_hbm.at[pl.ds(7812 * 32, 16)])


BPW = B_TOTAL // NW
CHUNK = 640
NBUF = 4
NCHUNK = BPW // CHUNK
ROUNDS = NCHUNK // NBUF


@functools.partial(
    pl.kernel,
    mesh=_mesh,
    out_type=jax.ShapeDtypeStruct((B_TOTAL, E), jnp.float32),
    scratch_types=[
        pltpu.VMEM((BPW,), jnp.int32),
        pltpu.VMEM((NBUF, CHUNK, E), jnp.float32),
        pltpu.SemaphoreType.DMA,
        pltpu.SemaphoreType.DMA,
        pltpu.SemaphoreType.DMA,
        pltpu.SemaphoreType.DMA,
        pltpu.SemaphoreType.DMA,
        pltpu.SemaphoreType.DMA,
        pltpu.SemaphoreType.DMA,
        pltpu.SemaphoreType.DMA,
    ],
    compiler_params=pltpu.CompilerParams(use_tc_tiling_on_sc=False),
)
def _sc_gather(idx_hbm, table_hbm, out_hbm, idx_v, rows_v,
               g0, g1, g2, g3, w0, w1, w2, w3):
    gsem = (g0, g1, g2, g3)
    wsem = (w0, w1, w2, w3)
    wid = lax.axis_index("s") * NUM_CORES + lax.axis_index("c")
    base = wid * BPW

    pltpu.sync_copy(idx_hbm.at[pl.ds(base, BPW)], idx_v)

    def start_gather(chunk, buf):
        idx_slice = idx_v.at[pl.ds(chunk * CHUNK, CHUNK)]
        pltpu.async_copy(table_hbm.at[idx_slice], rows_v.at[buf], gsem[buf])

    for b in range(NBUF - 1):
        start_gather(b, b)

    @pl.loop(0, ROUNDS)
    def _round(r):
        for b in range(NBUF):
            c = r * NBUF + b
            bg = (b + NBUF - 1) % NBUF
            if b == 0:
                @pl.when(r > 0)
                def _():
                    pltpu.make_async_copy(
                        rows_v.at[bg], out_hbm.at[pl.ds(0, CHUNK)],
                        wsem[bg]).wait()
                start_gather(c + NBUF - 1, bg)
            else:
                pltpu.make_async_copy(
                    rows_v.at[bg], out_hbm.at[pl.ds(0, CHUNK)],
                    wsem[bg]).wait()

                @pl.when(r < ROUNDS - 1)
                def _():
                    start_gather(c + NBUF - 1, bg)
            pltpu.make_async_copy(
                table_hbm.at[idx_v.at[pl.ds(0, CHUNK)]], rows_v.at[b],
                gsem[b]).wait()
            pltpu.async_copy(
                rows_v.at[b], out_hbm.at[pl.ds(base + c * CHUNK, CHUNK)],
                wsem[b])

    pltpu.make_async_copy(
        rows_v.at[NBUF - 1], out_hbm.at[pl.ds(0, CHUNK)],
        wsem[NBUF - 1]).wait()


C_UNITS = L_SEQ * (B_BATCH // 512)
C_UPW = C_UNITS // NW


@functools.partial(
    pl.kernel,
    mesh=_mesh,
    out_type=jax.ShapeDtypeStruct((L_SEQ, 1, E, B_BATCH), jnp.float32),
    scratch_types=[
        pltpu.VMEM((2, 128, 128), jnp.float32),
        pltpu.VMEM((2, E, 512), jnp.float32),
        pltpu.SemaphoreType.DMA,
        pltpu.SemaphoreType.DMA,
        pltpu.SemaphoreType.DMA,
        pltpu.SemaphoreType.DMA,
    ],
    compiler_params=pltpu.CompilerParams(needs_layout_passes=False),
)
def _reformat_out(rows_hbm, out_hbm, in_v, out_v, gi0, gi1, wo0, wo1):
    gsem = (gi0, gi1)
    wsem = (wo0, wo1)
    wid = lax.axis_index("s") * NUM_CORES + lax.axis_index("c")

    def unit_of(k):
        return wid * C_UPW + k

    def start_in(k, b):
        u = unit_of(k)
        pltpu.async_copy(
            rows_hbm.at[pl.ds(u * 128, 128)], in_v.at[b], gsem[b])

    start_in(0, 0)

    @pl.loop(0, C_UPW // 2)
    def _pair(p):
        for b in range(2):
            k = p * 2 + b
            u = unit_of(k)
            l = u // (B_BATCH // 512)
            jb = lax.rem(u, B_BATCH // 512)
            pltpu.make_async_copy(
                rows_hbm.at[pl.ds(0, 128)], in_v.at[b], gsem[b]).wait()

            @pl.when(k + 1 < C_UPW)
            def _():
                start_in(k + 1, 1 - b)

            @pl.when(k >= 2)
            def _():
                pltpu.make_async_copy(
                    out_v.at[b], out_hbm.at[0, 0, :, pl.ds(0, 512)],
                    wsem[b]).wait()

            @pl.loop(0, 4)
            def _col(c):
                _untranspose_block(in_v.at[b], out_v.at[b], c * 32,
                                   c * 128, 8)

            pltpu.async_copy(
                out_v.at[b], out_hbm.at[l, 0, :, pl.ds(jb * 512, 512)],
                wsem[b])

    for b in range(2):
        pltpu.make_async_copy(
            out_v.at[b], out_hbm.at[0, 0, :, pl.ds(0, 512)], wsem[b]).wait()


def kernel(x, vocab):
    vt = jnp.transpose(vocab, (1, 2, 0))
    tail = vocab[4 * (ROWS4 - 16):, 0, :].reshape(16, 128)
    table4 = _reformat_table(vt, tail)
    table = table4.reshape(V_SIZE, E)
    idx = jnp.transpose(x).reshape(-1).astype(jnp.int32)
    rows = _sc_gather(idx, table)
    rows4 = rows.reshape(B_TOTAL // 4, 128)
    out = _reformat_out(rows4)
    return jnp.transpose(out, (3, 0, 1, 2))
